# Initial kernel scaffold; baseline (speedup 1.0000x reference)
import jax
import jax.numpy as jnp
from jax.experimental import pallas as pl


def kernel(nodes, ui_network, ratings, u_weight, i_weight, W1, b1, W2, b2, W3, b3):
    raise NotImplementedError("write your pallas kernel here")


# speedup vs baseline: 1.0685x; 1.0685x over previous
import functools

import jax
import jax.numpy as jnp
from jax import lax
from jax.experimental import pallas as pl
from jax.experimental.pallas import tpu as pltpu
from jax.experimental.pallas import tpu_sc as plsc

B = 16384
L = 20
D = 32

_NC = 2
_NS = 16
_NW = _NC * _NS

_NEIGH_PW = (B * L) // _NW
_NODE_PW = B // _NW
_CHUNK = 2048
_NCHUNKS = _NEIGH_PW // _CHUNK


def _sc_gather(i_weight, u_weight, ui_flat, nodes):
    mesh = plsc.VectorSubcoreMesh(core_axis_name="c", subcore_axis_name="s")

    @functools.partial(
        pl.kernel,
        mesh=mesh,
        out_type=[
            jax.ShapeDtypeStruct((B * L, D), jnp.float32),
            jax.ShapeDtypeStruct((B, D), jnp.float32),
        ],
        scratch_types=[
            pltpu.VMEM((_CHUNK,), jnp.int32),
            pltpu.VMEM((_CHUNK, D), jnp.float32),
            pltpu.VMEM((_NODE_PW,), jnp.int32),
            pltpu.VMEM((_NODE_PW, D), jnp.float32),
            pltpu.SemaphoreType.DMA,
        ],
        compiler_params=pltpu.CompilerParams(use_tc_tiling_on_sc=False),
    )
    def k(iw_hbm, uw_hbm, ui_hbm, nodes_hbm, neigh_out, node_out,
          idx_v, rows_v, nidx_v, nrows_v, sem):
        wid = lax.axis_index("s") * _NC + lax.axis_index("c")
        nb = wid * _NEIGH_PW
        for c in range(_NCHUNKS):
            base = nb + c * _CHUNK
            pltpu.sync_copy(ui_hbm.at[pl.ds(base, _CHUNK)], idx_v)
            pltpu.async_copy(iw_hbm.at[idx_v], rows_v, sem).wait()
            pltpu.sync_copy(rows_v, neigh_out.at[pl.ds(base, _CHUNK)])
        nbase = wid * _NODE_PW
        pltpu.sync_copy(nodes_hbm.at[pl.ds(nbase, _NODE_PW)], nidx_v)
        pltpu.async_copy(uw_hbm.at[nidx_v], nrows_v, sem).wait()
        pltpu.sync_copy(nrows_v, node_out.at[pl.ds(nbase, _NODE_PW)])

    return k(i_weight, u_weight, ui_flat, nodes)


_BT = 256


def _dense_body(neigh_ref, node_ref, w1n_ref, w1c_ref, b1_ref, w2_ref,
                b2_ref, w3_ref, out_ref):
    neigh = neigh_ref[...]
    node = node_ref[...]
    c1 = jnp.dot(node, w1c_ref[...],
                 preferred_element_type=jnp.float32) + b1_ref[...]
    h1 = jnp.dot(neigh, w1n_ref[...], preferred_element_type=jnp.float32)
    h1 = jnp.maximum(h1.reshape(_BT, L, D) + c1[:, None, :], 0.0)
    h2 = jnp.dot(h1.reshape(_BT * L, D), w2_ref[...],
                 preferred_element_type=jnp.float32) + b2_ref[...]
    h2 = jnp.maximum(h2, 0.0)
    logits = jnp.sum(h2.reshape(_BT, L, D) * w3_ref[...].reshape(1, 1, D),
                     axis=2)
    m = jnp.max(logits, axis=1, keepdims=True)
    e = jnp.exp(logits - m)
    att = e / jnp.sum(e, axis=1, keepdims=True)
    out_ref[...] = jnp.sum(neigh.reshape(_BT, L, D) * att[:, :, None], axis=1)


def _tc_dense(neighs, node_emb, w1n, w1c, b1, w2, b2, w3):
    grid = (B // _BT,)
    return pl.pallas_call(
        _dense_body,
        grid=grid,
        in_specs=[
            pl.BlockSpec((_BT * L, D), lambda i: (i, 0)),
            pl.BlockSpec((_BT, D), lambda i: (i, 0)),
            pl.BlockSpec((D, D), lambda i: (0, 0)),
            pl.BlockSpec((D, D), lambda i: (0, 0)),
            pl.BlockSpec((1, D), lambda i: (0, 0)),
            pl.BlockSpec((D, D), lambda i: (0, 0)),
            pl.BlockSpec((1, D), lambda i: (0, 0)),
            pl.BlockSpec((1, D), lambda i: (0, 0)),
        ],
        out_specs=pl.BlockSpec((_BT, D), lambda i: (i, 0)),
        out_shape=jax.ShapeDtypeStruct((B, D), jnp.float32),
        compiler_params=pltpu.CompilerParams(
            dimension_semantics=("arbitrary",)),
    )(neighs, node_emb, w1n, w1c, b1, w2, b2, w3)


def kernel(nodes, ui_network, ratings, u_weight, i_weight, W1, b1, W2, b2, W3, b3):
    ui_flat = ui_network.reshape(-1).astype(jnp.int32)
    nodes32 = nodes.astype(jnp.int32)
    neighs, node_emb = _sc_gather(i_weight, u_weight, ui_flat, nodes32)
    w1n = W1[:, :D].T
    w1c = W1[:, D:].T
    w2 = W2.T
    return _tc_dense(neighs, node_emb, w1n, w1c, b1.reshape(1, D),
                     w2, b2.reshape(1, D), W3.reshape(1, D))
